# baseline (speedup 1.0000x reference)
import functools

import jax
import jax.numpy as jnp
from jax import lax
from jax.experimental import pallas as pl
from jax.experimental.pallas import tpu as pltpu
from jax.experimental.pallas import tpu_sc as plsc

_NUM_CORES = 2
_NUM_SUBCORES = 16
_LANES = 16
_NUM_WORKERS = _NUM_CORES * _NUM_SUBCORES

_LATENT = 64
_IDX_CHUNK = 128


def _perm(x, idx):
    return lax.gather(
        x, idx[:, None],
        dimension_numbers=lax.GatherDimensionNumbers(
            offset_dims=(), collapsed_slice_dims=(0,), start_index_map=(0,)),
        slice_sizes=(1,),
        mode=lax.GatherScatterMode.PROMISE_IN_BOUNDS)


def _make_kernel(batch: int):
    b_per_w = batch // _NUM_WORKERS
    n_chunks = b_per_w // _IDX_CHUNK
    n_groups = b_per_w // _LANES

    mesh = plsc.VectorSubcoreMesh(
        core_axis_name="c",
        subcore_axis_name="s",
        num_cores=_NUM_CORES,
        num_subcores=_NUM_SUBCORES,
    )

    @functools.partial(
        pl.kernel,
        mesh=mesh,
        out_type=jax.ShapeDtypeStruct((batch,), jnp.float32),
        scratch_types=[
            pltpu.VMEM((n_chunks, _IDX_CHUNK), jnp.int32),
            pltpu.VMEM((n_chunks, _IDX_CHUNK), jnp.int32),
            pltpu.VMEM((b_per_w, _LATENT), jnp.float32),
            pltpu.VMEM((b_per_w, _LATENT), jnp.float32),
            pltpu.VMEM((b_per_w,), jnp.float32),
            pltpu.SemaphoreType.DMA,
        ],
        compiler_params=pltpu.CompilerParams(use_tc_tiling_on_sc=False),
    )
    def k(uidx_hbm, pidx_hbm, ut_hbm, pt_hbm, out_hbm,
          uidx_v, pidx_v, urows, prows, outv, sem):
        wid = lax.axis_index("s") * _NUM_CORES + lax.axis_index("c")
        base = wid * b_per_w

        for j in range(n_chunks):
            pltpu.sync_copy(
                uidx_hbm.at[pl.ds(base + j * _IDX_CHUNK, _IDX_CHUNK)],
                uidx_v.at[j])
            pltpu.sync_copy(
                pidx_hbm.at[pl.ds(base + j * _IDX_CHUNK, _IDX_CHUNK)],
                pidx_v.at[j])

        copies = []
        for j in range(n_chunks):
            copies.append(pltpu.async_copy(
                ut_hbm.at[uidx_v.at[j]],
                urows.at[pl.ds(j * _IDX_CHUNK, _IDX_CHUNK)], sem))
            copies.append(pltpu.async_copy(
                pt_hbm.at[pidx_v.at[j]],
                prows.at[pl.ds(j * _IDX_CHUNK, _IDX_CHUNK)], sem))
        for cp in copies:
            cp.wait()

        lane = lax.iota(jnp.int32, 16)

        def group_body(g, _):
            res = jnp.zeros((_LANES,), jnp.float32)
            for j in range(_LANES):
                r = g * _LANES + j
                acc = jnp.zeros((_LANES,), jnp.float32)
                for c in range(_LATENT // _LANES):
                    u = urows[r, pl.ds(c * _LANES, _LANES)]
                    p = prows[r, pl.ds(c * _LANES, _LANES)]
                    acc = acc + u * p
                for step in (8, 4, 2, 1):
                    acc = acc + _perm(acc, lane ^ step)
                res = jnp.where(lane == j, acc, res)
            res = 1.0 / (1.0 + jnp.exp(-res))
            outv[pl.ds(g * _LANES, _LANES)] = res
            return 0

        lax.fori_loop(0, n_groups, group_body, 0)

        pltpu.sync_copy(outv, out_hbm.at[pl.ds(base, b_per_w)])

    return k


@jax.jit
def kernel(inputs, user_table, product_table):
    batch = inputs.shape[0]
    uidx = inputs[:, 0].astype(jnp.int32)
    pidx = inputs[:, 1].astype(jnp.int32)
    k = _make_kernel(batch)
    return k(uidx, pidx, user_table, product_table)
